# initial kernel scaffold (unmeasured)
import jax
import jax.numpy as jnp
from jax import lax
from jax.experimental import pallas as pl
from jax.experimental.pallas import tpu as pltpu

N_DEV = 4
B = 32
H = 16
D = 128
BS = 32
NB_GLOBAL = 256
NP_LOCAL = 256
PG_BLK = 32
NKB = NP_LOCAL // PG_BLK
TOK_BLK = PG_BLK * BS
SCALE = D ** -0.5


def kernel(Q, K, V, bt, lens):
    lens2 = lens.reshape(B, 1)

    def body(q_ref, k_ref, v_ref, bt_ref, lens_ref, out_ref,
             counts_ref, e_ref, m_ref, l_ref, acc_ref, obuf, mlbuf,
             send_o, recv_o, send_ml, recv_ml):
        h = pl.program_id(0)
        b = pl.program_id(1)
        my = lax.axis_index("i")

        @pl.when((h == 0) & (b == 0))
        def _():
            base = my * NP_LOCAL
            pages = base + lax.broadcasted_iota(jnp.int32, (1, 1, NP_LOCAL), 2)
            lens_b = lens_ref[:, :].reshape(B, 1, 1)
            cnt = jnp.zeros((B, NP_LOCAL), jnp.float32)
            for jc in range(NB_GLOBAL // 32):
                btc = bt_ref[:, jc * 32:(jc + 1) * 32]
                jids = jc * 32 + lax.broadcasted_iota(jnp.int32, (1, 32, 1), 1)
                eq = (btc[:, :, None] == pages) & (jids < lens_b)
                cnt = cnt + jnp.sum(eq.astype(jnp.float32), axis=1)
            counts_ref[:, :] = cnt
            p_idx = lax.broadcasted_iota(jnp.int32, (PG_BLK, TOK_BLK), 0)
            t_idx = lax.broadcasted_iota(jnp.int32, (PG_BLK, TOK_BLK), 1)
            e_ref[:, :] = (t_idx // BS == p_idx).astype(jnp.bfloat16)

        @pl.when(b == 0)
        def _():
            m_ref[:, :] = jnp.full((B, 1), -jnp.inf, jnp.float32)
            l_ref[:, :] = jnp.zeros((B, 1), jnp.float32)
            acc_ref[:, :] = jnp.zeros((B, D), jnp.float32)

        q = q_ref[:, 0, 0, :].astype(jnp.bfloat16)
        k = k_ref[:, :, 0, :].reshape(TOK_BLK, D).astype(jnp.bfloat16)
        s = lax.dot_general(q, k, (((1,), (1,)), ((), ())),
                            preferred_element_type=jnp.float32)
        s = s * SCALE

        cblk = counts_ref[:, pl.ds(b * PG_BLK, PG_BLK)]
        ctok = lax.dot_general(cblk.astype(jnp.bfloat16), e_ref[:, :],
                               (((1,), (0,)), ((), ())),
                               preferred_element_type=jnp.float32)
        active = ctok > 0.0
        s = jnp.where(active, s, -jnp.inf)

        m_prev = m_ref[:, :]
        m_new = jnp.maximum(m_prev, jnp.max(s, axis=1, keepdims=True))
        alpha = jnp.where(m_new == -jnp.inf, 0.0, jnp.exp(m_prev - m_new))
        p = jnp.where(active, ctok * jnp.exp(s - m_new), 0.0)

        l_ref[:, :] = alpha * l_ref[:, :] + jnp.sum(p, axis=1, keepdims=True)
        v = v_ref[:, :, 0, :].reshape(TOK_BLK, D).astype(jnp.bfloat16)
        pv = lax.dot_general(p.astype(jnp.bfloat16), v, (((1,), (0,)), ((), ())),
                             preferred_element_type=jnp.float32)
        acc_ref[:, :] = alpha * acc_ref[:, :] + pv
        m_ref[:, :] = m_new

        @pl.when(b == NKB - 1)
        def _():
            obuf[0, :, pl.ds(h, 1), :] = acc_ref[:, :].reshape(B, 1, D)
            mlbuf[0, 0, :, pl.ds(h, 1)] = m_ref[:, :]
            mlbuf[0, 1, :, pl.ds(h, 1)] = l_ref[:, :]

        @pl.when((h == H - 1) & (b == NKB - 1))
        def _():
            rdmas = []
            for off in range(1, N_DEV):
                dst = lax.rem(my + off, N_DEV)
                so = pltpu.make_async_remote_copy(
                    src_ref=obuf.at[0], dst_ref=obuf.at[off],
                    send_sem=send_o.at[off - 1], recv_sem=recv_o.at[off - 1],
                    device_id=(dst,), device_id_type=pl.DeviceIdType.MESH)
                so.start()
                sml = pltpu.make_async_remote_copy(
                    src_ref=mlbuf.at[0], dst_ref=mlbuf.at[off],
                    send_sem=send_ml.at[off - 1], recv_sem=recv_ml.at[off - 1],
                    device_id=(dst,), device_id_type=pl.DeviceIdType.MESH)
                sml.start()
                rdmas.append((so, sml))
            for so, sml in rdmas:
                so.wait_recv()
                sml.wait_recv()

            m_all = mlbuf[:, 0, :, :]
            l_all = mlbuf[:, 1, :, :]
            m_g = jnp.max(m_all, axis=0)
            scale = jnp.where(m_all == -jnp.inf, 0.0,
                              jnp.exp(m_all - m_g[None, :, :]))
            l_g = jnp.sum(l_all * scale, axis=0)
            o_g = jnp.sum(obuf[:, :, :, :] * scale[:, :, :, None], axis=0)
            out_ref[:, :, :, :] = (o_g / l_g[:, :, None]).reshape(B, 1, H, D)

            for so, sml in rdmas:
                so.wait_send()
                sml.wait_send()

    grid = (H, NKB)
    return pl.pallas_call(
        body,
        grid=grid,
        in_specs=[
            pl.BlockSpec((B, 1, 1, D), lambda h, b: (0, 0, h, 0)),
            pl.BlockSpec((PG_BLK, BS, 1, D), lambda h, b: (b, 0, h, 0)),
            pl.BlockSpec((PG_BLK, BS, 1, D), lambda h, b: (b, 0, h, 0)),
            pl.BlockSpec((B, NB_GLOBAL), lambda h, b: (0, 0)),
            pl.BlockSpec((B, 1), lambda h, b: (0, 0)),
        ],
        out_specs=pl.BlockSpec((B, 1, H, D), lambda h, b: (0, 0, 0, 0)),
        out_shape=jax.ShapeDtypeStruct((B, 1, H, D), jnp.float32),
        scratch_shapes=[
            pltpu.VMEM((B, NP_LOCAL), jnp.float32),
            pltpu.VMEM((PG_BLK, TOK_BLK), jnp.bfloat16),
            pltpu.VMEM((B, 1), jnp.float32),
            pltpu.VMEM((B, 1), jnp.float32),
            pltpu.VMEM((B, D), jnp.float32),
            pltpu.VMEM((N_DEV, B, H, D), jnp.float32),
            pltpu.VMEM((N_DEV, 2, B, H), jnp.float32),
            pltpu.SemaphoreType.DMA((N_DEV - 1,)),
            pltpu.SemaphoreType.DMA((N_DEV - 1,)),
            pltpu.SemaphoreType.DMA((N_DEV - 1,)),
            pltpu.SemaphoreType.DMA((N_DEV - 1,)),
        ],
        compiler_params=pltpu.CompilerParams(
            dimension_semantics=("arbitrary", "arbitrary"),
        ),
    )(Q, K, V, bt, lens2)


# baseline (device time: 253392 ns/iter reference)
import jax
import jax.numpy as jnp
from jax import lax
from jax.experimental import pallas as pl
from jax.experimental.pallas import tpu as pltpu

N_DEV = 4
B = 32
H = 16
D = 128
BS = 32
NB_GLOBAL = 256
NP_LOCAL = 256
PG_BLK = 32
NKB = NP_LOCAL // PG_BLK
TOK_BLK = PG_BLK * BS
SCALE = D ** -0.5


def kernel(Q, K, V, bt, lens):
    q2 = Q.reshape(B, H * D)
    k2 = K.reshape(NP_LOCAL * BS, H * D)
    v2 = V.reshape(NP_LOCAL * BS, H * D)
    lens2 = lens.reshape(B, 1)

    def body(q_ref, k_ref, v_ref, bt_ref, lens_ref, out_ref,
             counts_ref, e_ref, m_ref, l_ref, acc_ref, obuf, mlbuf,
             send_o, recv_o, send_ml, recv_ml):
        h = pl.program_id(0)
        b = pl.program_id(1)
        my = lax.axis_index("i")

        @pl.when((h == 0) & (b == 0))
        def _():
            lens_b = lens_ref[:, :].reshape(B, 1, 1)
            jids = lax.broadcasted_iota(jnp.int32, (1, NB_GLOBAL, 1), 1)
            valid = jids < lens_b
            bt_all = bt_ref[:, :]
            for kb in range(NKB):
                base = my * NP_LOCAL + kb * PG_BLK
                pages = base + lax.broadcasted_iota(
                    jnp.int32, (1, 1, PG_BLK), 2)
                eq = (bt_all[:, :, None] == pages) & valid
                counts_ref[kb, :, :] = jnp.sum(
                    eq.astype(jnp.float32), axis=1)
            p_idx = lax.broadcasted_iota(jnp.int32, (PG_BLK, TOK_BLK), 0)
            t_idx = lax.broadcasted_iota(jnp.int32, (PG_BLK, TOK_BLK), 1)
            e_ref[:, :] = (t_idx // BS == p_idx).astype(jnp.bfloat16)

        @pl.when(b == 0)
        def _():
            m_ref[:, :] = jnp.full((B, 1), -jnp.inf, jnp.float32)
            l_ref[:, :] = jnp.zeros((B, 1), jnp.float32)
            acc_ref[:, :] = jnp.zeros((B, D), jnp.float32)

        q = q_ref[:, :].astype(jnp.bfloat16)
        k = k_ref[:, :].astype(jnp.bfloat16)
        s = lax.dot_general(q, k, (((1,), (1,)), ((), ())),
                            preferred_element_type=jnp.float32)
        s = s * SCALE

        cblk = counts_ref[b]
        ctok = lax.dot_general(cblk.astype(jnp.bfloat16), e_ref[:, :],
                               (((1,), (0,)), ((), ())),
                               preferred_element_type=jnp.float32)
        active = ctok > 0.0
        s = jnp.where(active, s, -jnp.inf)

        m_prev = m_ref[:, :]
        m_new = jnp.maximum(m_prev, jnp.max(s, axis=1, keepdims=True))
        alpha = jnp.where(m_new == -jnp.inf, 0.0, jnp.exp(m_prev - m_new))
        p = jnp.where(active, ctok * jnp.exp(s - m_new), 0.0)

        l_ref[:, :] = alpha * l_ref[:, :] + jnp.sum(p, axis=1, keepdims=True)
        v = v_ref[:, :].astype(jnp.bfloat16)
        pv = lax.dot_general(p.astype(jnp.bfloat16), v, (((1,), (0,)), ((), ())),
                             preferred_element_type=jnp.float32)
        acc_ref[:, :] = alpha * acc_ref[:, :] + pv
        m_ref[:, :] = m_new

        @pl.when(b == NKB - 1)
        def _():
            obuf[0, h] = acc_ref[:, :]
            mlbuf[0, h] = m_ref[:, :]
            mlbuf[0, H + h] = l_ref[:, :]

        @pl.when((h == H - 1) & (b == NKB - 1))
        def _():
            rdmas = []
            for off in range(1, N_DEV):
                dst = lax.rem(my + off, N_DEV)
                so = pltpu.make_async_remote_copy(
                    src_ref=obuf.at[0], dst_ref=obuf.at[off],
                    send_sem=send_o.at[off - 1], recv_sem=recv_o.at[off - 1],
                    device_id=(dst,), device_id_type=pl.DeviceIdType.MESH)
                so.start()
                sml = pltpu.make_async_remote_copy(
                    src_ref=mlbuf.at[0], dst_ref=mlbuf.at[off],
                    send_sem=send_ml.at[off - 1], recv_sem=recv_ml.at[off - 1],
                    device_id=(dst,), device_id_type=pl.DeviceIdType.MESH)
                sml.start()
                rdmas.append((so, sml))
            for so, sml in rdmas:
                so.wait_recv()
                sml.wait_recv()

            m_all = mlbuf[:, :H]
            l_all = mlbuf[:, H:]
            m_g = jnp.max(m_all, axis=0)
            scale = jnp.where(m_all == -jnp.inf, 0.0,
                              jnp.exp(m_all - m_g[None]))
            l_g = jnp.sum(l_all * scale, axis=0)
            o_g = jnp.sum(obuf[:, :, :, :] * scale, axis=0)
            out_ref[:, :, :] = o_g / l_g

            for so, sml in rdmas:
                so.wait_send()
                sml.wait_send()

    grid = (H, NKB)
    out = pl.pallas_call(
        body,
        grid=grid,
        in_specs=[
            pl.BlockSpec((B, D), lambda h, b: (0, h)),
            pl.BlockSpec((TOK_BLK, D), lambda h, b: (b, h)),
            pl.BlockSpec((TOK_BLK, D), lambda h, b: (b, h)),
            pl.BlockSpec((B, NB_GLOBAL), lambda h, b: (0, 0)),
            pl.BlockSpec((B, 1), lambda h, b: (0, 0)),
        ],
        out_specs=pl.BlockSpec((H, B, D), lambda h, b: (0, 0, 0)),
        out_shape=jax.ShapeDtypeStruct((H, B, D), jnp.float32),
        scratch_shapes=[
            pltpu.VMEM((NKB, B, PG_BLK), jnp.float32),
            pltpu.VMEM((PG_BLK, TOK_BLK), jnp.bfloat16),
            pltpu.VMEM((B, 1), jnp.float32),
            pltpu.VMEM((B, 1), jnp.float32),
            pltpu.VMEM((B, D), jnp.float32),
            pltpu.VMEM((N_DEV, H, B, D), jnp.float32),
            pltpu.VMEM((N_DEV, 2 * H, B, 1), jnp.float32),
            pltpu.SemaphoreType.DMA((N_DEV - 1,)),
            pltpu.SemaphoreType.DMA((N_DEV - 1,)),
            pltpu.SemaphoreType.DMA((N_DEV - 1,)),
            pltpu.SemaphoreType.DMA((N_DEV - 1,)),
        ],
        compiler_params=pltpu.CompilerParams(
            dimension_semantics=("arbitrary", "arbitrary"),
        ),
    )(q2, k2, v2, bt, lens2)
    return jnp.transpose(out, (1, 0, 2)).reshape(B, 1, H, D)
